# baseline (speedup 1.0000x reference)
import jax
import jax.numpy as jnp
from jax.experimental import pallas as pl
from jax.experimental.pallas import tpu as pltpu

B = 256
T = 4
D_IN = 768
D_SAE = 16384
K_TOTAL = 128

BN = 512
NB = D_SAE // BN
CHUNK_TK = 1024


PRE_COLS = 4096


def _count_ge(pre_vmem, mid, ncols):
    acc = jnp.zeros((B, CHUNK_TK), jnp.int32)
    for c in range(ncols // CHUNK_TK):
        ch = jax.lax.bitcast_convert_type(
            pre_vmem[:, c * CHUNK_TK:(c + 1) * CHUNK_TK], jnp.int32)
        acc = acc + (ch >= mid).astype(jnp.int32)
    return jnp.sum(acc, axis=1, keepdims=True)


def _bisect_step(pre_vmem, carry, ncols):
    lo, hi = carry
    mid = lo + ((hi - lo) >> 1)
    cnt = _count_ge(pre_vmem, mid, ncols)
    take = cnt >= K_TOTAL
    return jnp.where(take, mid, lo), jnp.where(take, hi, mid)


def _bisect_tau(pre_vmem, lo0, hi0, tau_vmem):
    def cond(carry):
        lo, hi = carry
        return jnp.any(hi > lo + 1)

    def body(carry):
        return _bisect_step(pre_vmem, carry, D_SAE)

    lo, _ = jax.lax.while_loop(cond, body, (lo0, hi0))
    tau_vmem[...] = lo


def _fused_kernel(x_ref, we_ref, be_ref, wd_ref, bd_ref,
                  xhat_ref, z_ref, pre_vmem, tau_vmem, rmax_vmem, brk_vmem):
    j = pl.program_id(0)

    @pl.when(j < NB)
    def _encode():
        acc = jnp.dot(x_ref[...], we_ref[...], preferred_element_type=jnp.float32)
        acc = acc + be_ref[...]
        blk = jnp.where(acc > 0.0, acc, 0.0)
        pre_vmem[:, pl.ds(j * BN, BN)] = blk
        bmax = jnp.max(blk, axis=1, keepdims=True)

        @pl.when(j == 0)
        def _():
            rmax_vmem[...] = bmax

        @pl.when(j > 0)
        def _():
            rmax_vmem[...] = jnp.maximum(rmax_vmem[...], bmax)

    npre = PRE_COLS // BN

    @pl.when(j == npre)
    def _brk_init():
        brk_vmem[:, 0:1] = jnp.zeros((B, 1), jnp.int32)
        brk_vmem[:, 1:2] = jax.lax.bitcast_convert_type(
            rmax_vmem[...], jnp.int32) + 1

    @pl.when((j >= npre) & (j < NB - 1))
    def _brk_step():
        lo, hi = _bisect_step(
            pre_vmem, (brk_vmem[:, 0:1], brk_vmem[:, 1:2]), PRE_COLS)
        brk_vmem[:, 0:1] = lo
        brk_vmem[:, 1:2] = hi

    @pl.when(j == NB - 1)
    def _tau():
        lo0 = brk_vmem[:, 0:1]
        hi0 = jax.lax.bitcast_convert_type(rmax_vmem[...], jnp.int32) + 1
        hi0 = jnp.maximum(hi0, lo0 + 1)
        _bisect_tau(pre_vmem, lo0, hi0, tau_vmem)

    @pl.when(j == NB)
    def _init_out():
        xhat_ref[...] = jnp.broadcast_to(
            bd_ref[...].reshape(1, T, D_IN), xhat_ref.shape)

    @pl.when(j >= NB)
    def _decode():
        vals = pre_vmem[:, pl.ds((j - NB) * BN, BN)]
        bits = jax.lax.bitcast_convert_type(vals, jnp.int32)
        keep = (bits >= tau_vmem[...]) & (vals > 0.0)
        zb = jnp.where(keep, vals, 0.0)
        z_ref[...] = zb
        zb16 = zb.astype(jnp.bfloat16)
        for t in range(T):
            acc = jnp.dot(zb16, wd_ref[t].astype(jnp.bfloat16),
                          preferred_element_type=jnp.float32)
            xhat_ref[:, t, :] += acc


@jax.jit
def kernel(x, W_enc, b_enc, W_dec, b_dec):
    x2 = x.reshape(B, T * D_IN)
    w_enc2 = W_enc.reshape(T * D_IN, D_SAE)
    b_enc2 = b_enc.reshape(1, D_SAE)

    x_hat, z = pl.pallas_call(
        _fused_kernel,
        grid=(2 * NB,),
        in_specs=[
            pl.BlockSpec((B, T * D_IN), lambda j: (0, 0)),
            pl.BlockSpec((T * D_IN, BN), lambda j: (0, jnp.minimum(j, NB - 1))),
            pl.BlockSpec((1, BN), lambda j: (0, jnp.minimum(j, NB - 1))),
            pl.BlockSpec((T, BN, D_IN), lambda j: (0, jnp.maximum(j - NB, 0), 0)),
            pl.BlockSpec((T, D_IN), lambda j: (0, 0)),
        ],
        out_specs=[
            pl.BlockSpec((B, T, D_IN), lambda j: (0, 0, 0)),
            pl.BlockSpec((B, BN), lambda j: (0, jnp.maximum(j - NB, 0))),
        ],
        out_shape=[
            jax.ShapeDtypeStruct((B, T, D_IN), jnp.float32),
            jax.ShapeDtypeStruct((B, D_SAE), jnp.float32),
        ],
        scratch_shapes=[
            pltpu.VMEM((B, D_SAE), jnp.float32),
            pltpu.VMEM((B, 1), jnp.int32),
            pltpu.VMEM((B, 1), jnp.float32),
            pltpu.VMEM((B, 2), jnp.int32),
        ],
    )(x2, w_enc2, b_enc2, W_dec, b_dec)

    return (x_hat, z)


# speedup vs baseline: 1.0084x; 1.0084x over previous
import jax
import jax.numpy as jnp
from jax.experimental import pallas as pl
from jax.experimental.pallas import tpu as pltpu

B = 256
T = 4
D_IN = 768
D_SAE = 16384
K_TOTAL = 128

BN = 512
NB = D_SAE // BN
CHUNK_TK = 1024


PRE_COLS = 4096


def _count_ge(pre_vmem, mid, ncols):
    acc = jnp.zeros((B, CHUNK_TK), jnp.int32)
    for c in range(ncols // CHUNK_TK):
        ch = jax.lax.bitcast_convert_type(
            pre_vmem[:, c * CHUNK_TK:(c + 1) * CHUNK_TK], jnp.int32)
        acc = acc + (ch >= mid).astype(jnp.int32)
    return jnp.sum(acc, axis=1, keepdims=True)


def _bisect_step(pre_vmem, carry, ncols):
    lo, hi = carry
    mid = lo + ((hi - lo) >> 1)
    cnt = _count_ge(pre_vmem, mid, ncols)
    take = cnt >= K_TOTAL
    return jnp.where(take, mid, lo), jnp.where(take, hi, mid)


def _bisect_tau(pre_vmem, lo0, hi0, tau_vmem):
    rng = jnp.max(hi0 - lo0).astype(jnp.float32)
    n_it = (jax.lax.bitcast_convert_type(rng, jnp.int32) >> 23) - 127 + 2

    def body(_, carry):
        return _bisect_step(pre_vmem, carry, D_SAE)

    lo, _ = jax.lax.fori_loop(0, n_it, body, (lo0, hi0))
    tau_vmem[...] = lo


def _fused_kernel(x_ref, we_ref, be_ref, wd_ref, bd_ref,
                  xhat_ref, z_ref, pre_vmem, tau_vmem, rmax_vmem, brk_vmem):
    j = pl.program_id(0)

    @pl.when(j < NB)
    def _encode():
        acc = jnp.dot(x_ref[...], we_ref[...], preferred_element_type=jnp.float32)
        acc = acc + be_ref[...]
        blk = jnp.where(acc > 0.0, acc, 0.0)
        pre_vmem[:, pl.ds(j * BN, BN)] = blk
        bmax = jnp.max(blk, axis=1, keepdims=True)

        @pl.when(j == 0)
        def _():
            rmax_vmem[...] = bmax

        @pl.when(j > 0)
        def _():
            rmax_vmem[...] = jnp.maximum(rmax_vmem[...], bmax)

    npre = PRE_COLS // BN

    @pl.when(j == npre)
    def _brk_init():
        brk_vmem[:, 0:1] = jnp.zeros((B, 1), jnp.int32)
        brk_vmem[:, 1:2] = jax.lax.bitcast_convert_type(
            rmax_vmem[...], jnp.int32) + 1

    @pl.when((j >= npre) & (j < NB - 1))
    def _brk_step():
        lo, hi = _bisect_step(
            pre_vmem, (brk_vmem[:, 0:1], brk_vmem[:, 1:2]), PRE_COLS)
        brk_vmem[:, 0:1] = lo
        brk_vmem[:, 1:2] = hi

    @pl.when(j == NB - 1)
    def _tau():
        lo0 = brk_vmem[:, 0:1]
        hi0 = jax.lax.bitcast_convert_type(rmax_vmem[...], jnp.int32) + 1
        hi0 = jnp.maximum(hi0, lo0 + 1)
        _bisect_tau(pre_vmem, lo0, hi0, tau_vmem)

    @pl.when(j == NB)
    def _init_out():
        xhat_ref[...] = jnp.broadcast_to(
            bd_ref[...].reshape(1, T, D_IN), xhat_ref.shape)

    @pl.when(j >= NB)
    def _decode():
        vals = pre_vmem[:, pl.ds((j - NB) * BN, BN)]
        bits = jax.lax.bitcast_convert_type(vals, jnp.int32)
        keep = (bits >= tau_vmem[...]) & (vals > 0.0)
        zb = jnp.where(keep, vals, 0.0)
        z_ref[...] = zb
        zb16 = zb.astype(jnp.bfloat16)
        for t in range(T):
            acc = jnp.dot(zb16, wd_ref[t].astype(jnp.bfloat16),
                          preferred_element_type=jnp.float32)
            xhat_ref[:, t, :] += acc


@jax.jit
def kernel(x, W_enc, b_enc, W_dec, b_dec):
    x2 = x.reshape(B, T * D_IN)
    w_enc2 = W_enc.reshape(T * D_IN, D_SAE)
    b_enc2 = b_enc.reshape(1, D_SAE)

    x_hat, z = pl.pallas_call(
        _fused_kernel,
        grid=(2 * NB,),
        in_specs=[
            pl.BlockSpec((B, T * D_IN), lambda j: (0, 0)),
            pl.BlockSpec((T * D_IN, BN), lambda j: (0, jnp.minimum(j, NB - 1))),
            pl.BlockSpec((1, BN), lambda j: (0, jnp.minimum(j, NB - 1))),
            pl.BlockSpec((T, BN, D_IN), lambda j: (0, jnp.maximum(j - NB, 0), 0)),
            pl.BlockSpec((T, D_IN), lambda j: (0, 0)),
        ],
        out_specs=[
            pl.BlockSpec((B, T, D_IN), lambda j: (0, 0, 0)),
            pl.BlockSpec((B, BN), lambda j: (0, jnp.maximum(j - NB, 0))),
        ],
        out_shape=[
            jax.ShapeDtypeStruct((B, T, D_IN), jnp.float32),
            jax.ShapeDtypeStruct((B, D_SAE), jnp.float32),
        ],
        scratch_shapes=[
            pltpu.VMEM((B, D_SAE), jnp.float32),
            pltpu.VMEM((B, 1), jnp.int32),
            pltpu.VMEM((B, 1), jnp.float32),
            pltpu.VMEM((B, 2), jnp.int32),
        ],
    )(x2, w_enc2, b_enc2, W_dec, b_dec)

    return (x_hat, z)


# speedup vs baseline: 1.0091x; 1.0007x over previous
import jax
import jax.numpy as jnp
from jax.experimental import pallas as pl
from jax.experimental.pallas import tpu as pltpu

B = 256
T = 4
D_IN = 768
D_SAE = 16384
K_TOTAL = 128

BN = 512
NB = D_SAE // BN
CHUNK_TK = 1024


PRE_COLS = 1024


def _count_ge(pre_vmem, mid, ncols):
    acc = jnp.zeros((B, CHUNK_TK), jnp.int32)
    for c in range(ncols // CHUNK_TK):
        ch = jax.lax.bitcast_convert_type(
            pre_vmem[:, c * CHUNK_TK:(c + 1) * CHUNK_TK], jnp.int32)
        acc = acc + (ch >= mid).astype(jnp.int32)
    return jnp.sum(acc, axis=1, keepdims=True)


def _bisect_step(pre_vmem, carry, ncols):
    lo, hi = carry
    mid = lo + ((hi - lo) >> 1)
    cnt = _count_ge(pre_vmem, mid, ncols)
    take = cnt >= K_TOTAL
    return jnp.where(take, mid, lo), jnp.where(take, hi, mid)


def _bisect_tau(pre_vmem, lo0, hi0, tau_vmem):
    rng = jnp.max(hi0 - lo0).astype(jnp.float32)
    n_it = (jax.lax.bitcast_convert_type(rng, jnp.int32) >> 23) - 127 + 2

    def body(_, carry):
        return _bisect_step(pre_vmem, carry, D_SAE)

    lo, _ = jax.lax.fori_loop(0, n_it, body, (lo0, hi0))
    tau_vmem[...] = lo


def _fused_kernel(x_ref, we_ref, be_ref, wd_ref, bd_ref,
                  xhat_ref, z_ref, pre_vmem, tau_vmem, rmax_vmem, brk_vmem):
    j = pl.program_id(0)

    @pl.when(j < NB)
    def _encode():
        acc = jnp.dot(x_ref[...], we_ref[...], preferred_element_type=jnp.float32)
        acc = acc + be_ref[...]
        blk = jnp.where(acc > 0.0, acc, 0.0)
        pre_vmem[:, pl.ds(j * BN, BN)] = blk
        bmax = jnp.max(blk, axis=1, keepdims=True)

        @pl.when(j == 0)
        def _():
            rmax_vmem[...] = bmax

        @pl.when(j > 0)
        def _():
            rmax_vmem[...] = jnp.maximum(rmax_vmem[...], bmax)

    npre = PRE_COLS // BN

    @pl.when(j == npre)
    def _brk_init():
        brk_vmem[:, 0:1] = jnp.zeros((B, 1), jnp.int32)
        brk_vmem[:, 1:2] = jax.lax.bitcast_convert_type(
            rmax_vmem[...], jnp.int32) + 1

    @pl.when((j >= npre) & (j < NB - 1))
    def _brk_step():
        lo, hi = _bisect_step(
            pre_vmem, (brk_vmem[:, 0:1], brk_vmem[:, 1:2]), PRE_COLS)
        brk_vmem[:, 0:1] = lo
        brk_vmem[:, 1:2] = hi

    @pl.when(j == NB - 1)
    def _tau():
        lo0 = brk_vmem[:, 0:1]
        hi0 = jax.lax.bitcast_convert_type(rmax_vmem[...], jnp.int32) + 1
        hi0 = jnp.maximum(hi0, lo0 + 1)
        _bisect_tau(pre_vmem, lo0, hi0, tau_vmem)

    @pl.when(j == NB)
    def _init_out():
        xhat_ref[...] = jnp.broadcast_to(
            bd_ref[...].reshape(1, T, D_IN), xhat_ref.shape)

    @pl.when(j >= NB)
    def _decode():
        vals = pre_vmem[:, pl.ds((j - NB) * BN, BN)]
        bits = jax.lax.bitcast_convert_type(vals, jnp.int32)
        keep = (bits >= tau_vmem[...]) & (vals > 0.0)
        zb = jnp.where(keep, vals, 0.0)
        z_ref[...] = zb
        zb16 = zb.astype(jnp.bfloat16)
        for t in range(T):
            acc = jnp.dot(zb16, wd_ref[t].astype(jnp.bfloat16),
                          preferred_element_type=jnp.float32)
            xhat_ref[:, t, :] += acc


@jax.jit
def kernel(x, W_enc, b_enc, W_dec, b_dec):
    x2 = x.reshape(B, T * D_IN)
    w_enc2 = W_enc.reshape(T * D_IN, D_SAE)
    b_enc2 = b_enc.reshape(1, D_SAE)

    x_hat, z = pl.pallas_call(
        _fused_kernel,
        grid=(2 * NB,),
        in_specs=[
            pl.BlockSpec((B, T * D_IN), lambda j: (0, 0)),
            pl.BlockSpec((T * D_IN, BN), lambda j: (0, jnp.minimum(j, NB - 1))),
            pl.BlockSpec((1, BN), lambda j: (0, jnp.minimum(j, NB - 1))),
            pl.BlockSpec((T, BN, D_IN), lambda j: (0, jnp.maximum(j - NB, 0), 0)),
            pl.BlockSpec((T, D_IN), lambda j: (0, 0)),
        ],
        out_specs=[
            pl.BlockSpec((B, T, D_IN), lambda j: (0, 0, 0)),
            pl.BlockSpec((B, BN), lambda j: (0, jnp.maximum(j - NB, 0))),
        ],
        out_shape=[
            jax.ShapeDtypeStruct((B, T, D_IN), jnp.float32),
            jax.ShapeDtypeStruct((B, D_SAE), jnp.float32),
        ],
        scratch_shapes=[
            pltpu.VMEM((B, D_SAE), jnp.float32),
            pltpu.VMEM((B, 1), jnp.int32),
            pltpu.VMEM((B, 1), jnp.float32),
            pltpu.VMEM((B, 2), jnp.int32),
        ],
    )(x2, w_enc2, b_enc2, W_dec, b_dec)

    return (x_hat, z)


# speedup vs baseline: 1.0606x; 1.0510x over previous
import jax
import jax.numpy as jnp
from jax.experimental import pallas as pl
from jax.experimental.pallas import tpu as pltpu

B = 256
T = 4
D_IN = 768
D_SAE = 16384
K_TOTAL = 128

BN = 512
NB = D_SAE // BN
CHUNK_TK = 1024


def _count_ge(pre_vmem, mid):
    acc = jnp.zeros((B, CHUNK_TK), jnp.int32)
    for c in range(D_SAE // CHUNK_TK):
        ch = jax.lax.bitcast_convert_type(
            pre_vmem[:, c * CHUNK_TK:(c + 1) * CHUNK_TK], jnp.int32)
        acc = acc + (ch >= mid).astype(jnp.int32)
    return jnp.sum(acc, axis=1, keepdims=True)


def _bisect_tau(pre_vmem, tau_vmem):
    def cond(carry):
        lo, hi = carry
        return jnp.any(hi > lo + 1)

    def body(carry):
        lo, hi = carry
        mid = lo + ((hi - lo) >> 1)
        cnt = _count_ge(pre_vmem, mid)
        take = cnt >= K_TOTAL
        lo = jnp.where(take, mid, lo)
        hi = jnp.where(take, hi, mid)
        exact = cnt == K_TOTAL
        hi = jnp.where(exact, lo + 1, hi)
        return lo, hi

    lo0 = jnp.zeros((B, 1), jnp.int32)
    hi0 = jnp.full((B, 1), jnp.int32(0x7FFFFFFF))
    lo, _ = jax.lax.while_loop(cond, body, (lo0, hi0))
    tau_vmem[...] = lo


def _fused_kernel(x_ref, we_ref, be_ref, wd_ref, bd_ref,
                  xhat_ref, z_ref, pre_vmem, tau_vmem):
    j = pl.program_id(0)

    @pl.when(j < NB)
    def _encode():
        acc = jnp.dot(x_ref[...], we_ref[...], preferred_element_type=jnp.float32)
        acc = acc + be_ref[...]
        pre_vmem[:, pl.ds(j * BN, BN)] = jnp.where(acc > 0.0, acc, 0.0)

    @pl.when(j == NB - 1)
    def _tau():
        _bisect_tau(pre_vmem, tau_vmem)

    @pl.when(j == NB)
    def _init_out():
        xhat_ref[...] = jnp.broadcast_to(
            bd_ref[...].reshape(1, T, D_IN), xhat_ref.shape)

    @pl.when(j >= NB)
    def _decode():
        vals = pre_vmem[:, pl.ds((j - NB) * BN, BN)]
        bits = jax.lax.bitcast_convert_type(vals, jnp.int32)
        keep = (bits >= tau_vmem[...]) & (vals > 0.0)
        zb = jnp.where(keep, vals, 0.0)
        z_ref[...] = zb
        zb16 = zb.astype(jnp.bfloat16)
        for t in range(T):
            acc = jnp.dot(zb16, wd_ref[t].astype(jnp.bfloat16),
                          preferred_element_type=jnp.float32)
            xhat_ref[:, t, :] += acc


@jax.jit
def kernel(x, W_enc, b_enc, W_dec, b_dec):
    x2 = x.reshape(B, T * D_IN)
    w_enc2 = W_enc.reshape(T * D_IN, D_SAE)
    b_enc2 = b_enc.reshape(1, D_SAE)

    x_hat, z = pl.pallas_call(
        _fused_kernel,
        grid=(2 * NB,),
        in_specs=[
            pl.BlockSpec((B, T * D_IN), lambda j: (0, 0)),
            pl.BlockSpec((T * D_IN, BN), lambda j: (0, jnp.minimum(j, NB - 1))),
            pl.BlockSpec((1, BN), lambda j: (0, jnp.minimum(j, NB - 1))),
            pl.BlockSpec((T, BN, D_IN), lambda j: (0, jnp.maximum(j - NB, 0), 0)),
            pl.BlockSpec((T, D_IN), lambda j: (0, 0)),
        ],
        out_specs=[
            pl.BlockSpec((B, T, D_IN), lambda j: (0, 0, 0)),
            pl.BlockSpec((B, BN), lambda j: (0, jnp.maximum(j - NB, 0))),
        ],
        out_shape=[
            jax.ShapeDtypeStruct((B, T, D_IN), jnp.float32),
            jax.ShapeDtypeStruct((B, D_SAE), jnp.float32),
        ],
        scratch_shapes=[
            pltpu.VMEM((B, D_SAE), jnp.float32),
            pltpu.VMEM((B, 1), jnp.int32),
        ],
    )(x2, w_enc2, b_enc2, W_dec, b_dec)

    return (x_hat, z)


# speedup vs baseline: 1.0608x; 1.0002x over previous
import jax
import jax.numpy as jnp
from jax.experimental import pallas as pl
from jax.experimental.pallas import tpu as pltpu

B = 256
T = 4
D_IN = 768
D_SAE = 16384
K_TOTAL = 128

BN = 512
NB = D_SAE // BN
CHUNK_TK = 1024


def _count_ge(pre_vmem, mid):
    acc = jnp.zeros((B, CHUNK_TK), jnp.int32)
    for c in range(D_SAE // CHUNK_TK):
        ch = jax.lax.bitcast_convert_type(
            pre_vmem[:, c * CHUNK_TK:(c + 1) * CHUNK_TK], jnp.int32)
        acc = acc + (ch >= mid).astype(jnp.int32)
    return jnp.sum(acc, axis=1, keepdims=True)


def _bisect_tau(pre_vmem, tau_vmem):
    def cond(carry):
        lo, hi = carry
        return jnp.any(hi > lo + 1)

    def body(carry):
        lo, hi = carry
        mid = lo + ((hi - lo) >> 1)
        cnt = _count_ge(pre_vmem, mid)
        take = cnt >= K_TOTAL
        lo = jnp.where(take, mid, lo)
        hi = jnp.where(take, hi, mid)
        exact = cnt == K_TOTAL
        hi = jnp.where(exact, lo + 1, hi)
        return lo, hi

    lo0 = jnp.zeros((B, 1), jnp.int32)
    hi0 = jnp.full((B, 1), jnp.int32(0x7FFFFFFF))
    carry = jax.lax.fori_loop(0, 14, lambda _, c: body(c), (lo0, hi0))
    lo, _ = jax.lax.while_loop(cond, body, carry)
    tau_vmem[...] = lo


def _fused_kernel(x_ref, we_ref, be_ref, wd_ref, bd_ref,
                  xhat_ref, z_ref, pre_vmem, tau_vmem):
    j = pl.program_id(0)

    @pl.when(j < NB)
    def _encode():
        acc = jnp.dot(x_ref[...], we_ref[...], preferred_element_type=jnp.float32)
        acc = acc + be_ref[...]
        pre_vmem[:, pl.ds(j * BN, BN)] = jnp.where(acc > 0.0, acc, 0.0)

    @pl.when(j == NB - 1)
    def _tau():
        _bisect_tau(pre_vmem, tau_vmem)

    @pl.when(j == NB)
    def _init_out():
        xhat_ref[...] = jnp.broadcast_to(
            bd_ref[...].reshape(1, T, D_IN), xhat_ref.shape)

    @pl.when(j >= NB)
    def _decode():
        vals = pre_vmem[:, pl.ds((j - NB) * BN, BN)]
        bits = jax.lax.bitcast_convert_type(vals, jnp.int32)
        keep = (bits >= tau_vmem[...]) & (vals > 0.0)
        zb = jnp.where(keep, vals, 0.0)
        z_ref[...] = zb
        zb16 = zb.astype(jnp.bfloat16)
        for t in range(T):
            acc = jnp.dot(zb16, wd_ref[t].astype(jnp.bfloat16),
                          preferred_element_type=jnp.float32)
            xhat_ref[:, t, :] += acc


@jax.jit
def kernel(x, W_enc, b_enc, W_dec, b_dec):
    x2 = x.reshape(B, T * D_IN)
    w_enc2 = W_enc.reshape(T * D_IN, D_SAE)
    b_enc2 = b_enc.reshape(1, D_SAE)

    x_hat, z = pl.pallas_call(
        _fused_kernel,
        grid=(2 * NB,),
        in_specs=[
            pl.BlockSpec((B, T * D_IN), lambda j: (0, 0)),
            pl.BlockSpec((T * D_IN, BN), lambda j: (0, jnp.minimum(j, NB - 1))),
            pl.BlockSpec((1, BN), lambda j: (0, jnp.minimum(j, NB - 1))),
            pl.BlockSpec((T, BN, D_IN), lambda j: (0, jnp.maximum(j - NB, 0), 0)),
            pl.BlockSpec((T, D_IN), lambda j: (0, 0)),
        ],
        out_specs=[
            pl.BlockSpec((B, T, D_IN), lambda j: (0, 0, 0)),
            pl.BlockSpec((B, BN), lambda j: (0, jnp.maximum(j - NB, 0))),
        ],
        out_shape=[
            jax.ShapeDtypeStruct((B, T, D_IN), jnp.float32),
            jax.ShapeDtypeStruct((B, D_SAE), jnp.float32),
        ],
        scratch_shapes=[
            pltpu.VMEM((B, D_SAE), jnp.float32),
            pltpu.VMEM((B, 1), jnp.int32),
        ],
    )(x2, w_enc2, b_enc2, W_dec, b_dec)

    return (x_hat, z)


# speedup vs baseline: 1.0634x; 1.0025x over previous
import jax
import jax.numpy as jnp
from jax.experimental import pallas as pl
from jax.experimental.pallas import tpu as pltpu

B = 256
T = 4
D_IN = 768
D_SAE = 16384
K_TOTAL = 128

BN = 512
NB = D_SAE // BN
CHUNK_TK = 1024


def _count_ge(pre_vmem, mid):
    acc = jnp.zeros((B, CHUNK_TK), jnp.int32)
    for c in range(D_SAE // CHUNK_TK):
        ch = jax.lax.bitcast_convert_type(
            pre_vmem[:, c * CHUNK_TK:(c + 1) * CHUNK_TK], jnp.int32)
        acc = acc + (ch >= mid).astype(jnp.int32)
    return jnp.sum(acc, axis=1, keepdims=True)


def _bisect_tau(pre_vmem, tau_vmem):
    def cond(carry):
        lo, hi = carry
        return jnp.any(hi > lo + 1)

    def body(carry):
        lo, hi = carry
        mid = lo + ((hi - lo) >> 1)
        cnt = _count_ge(pre_vmem, mid)
        take = cnt >= K_TOTAL
        lo = jnp.where(take, mid, lo)
        hi = jnp.where(take, hi, mid)
        exact = cnt == K_TOTAL
        hi = jnp.where(exact, lo + 1, hi)
        return lo, hi

    lo0 = jnp.zeros((B, 1), jnp.int32)
    hi0 = jnp.full((B, 1), jnp.int32(0x7FFFFFFF))
    carry = jax.lax.fori_loop(0, 17, lambda _, c: body(c), (lo0, hi0))
    lo, _ = jax.lax.while_loop(cond, body, carry)
    tau_vmem[...] = lo


def _fused_kernel(x_ref, we_ref, be_ref, wd_ref, bd_ref,
                  xhat_ref, z_ref, pre_vmem, tau_vmem):
    j = pl.program_id(0)

    @pl.when(j < NB)
    def _encode():
        acc = jnp.dot(x_ref[...], we_ref[...], preferred_element_type=jnp.float32)
        acc = acc + be_ref[...]
        pre_vmem[:, pl.ds(j * BN, BN)] = jnp.where(acc > 0.0, acc, 0.0)

    @pl.when(j == NB - 1)
    def _tau():
        _bisect_tau(pre_vmem, tau_vmem)

    @pl.when(j == NB)
    def _init_out():
        xhat_ref[...] = jnp.broadcast_to(
            bd_ref[...].reshape(1, T, D_IN), xhat_ref.shape)

    @pl.when(j >= NB)
    def _decode():
        vals = pre_vmem[:, pl.ds((j - NB) * BN, BN)]
        bits = jax.lax.bitcast_convert_type(vals, jnp.int32)
        keep = (bits >= tau_vmem[...]) & (vals > 0.0)
        zb = jnp.where(keep, vals, 0.0)
        z_ref[...] = zb
        zb16 = zb.astype(jnp.bfloat16)
        for t in range(T):
            acc = jnp.dot(zb16, wd_ref[t].astype(jnp.bfloat16),
                          preferred_element_type=jnp.float32)
            xhat_ref[:, t, :] += acc


@jax.jit
def kernel(x, W_enc, b_enc, W_dec, b_dec):
    x2 = x.reshape(B, T * D_IN)
    w_enc2 = W_enc.reshape(T * D_IN, D_SAE)
    b_enc2 = b_enc.reshape(1, D_SAE)

    x_hat, z = pl.pallas_call(
        _fused_kernel,
        grid=(2 * NB,),
        in_specs=[
            pl.BlockSpec((B, T * D_IN), lambda j: (0, 0)),
            pl.BlockSpec((T * D_IN, BN), lambda j: (0, jnp.minimum(j, NB - 1))),
            pl.BlockSpec((1, BN), lambda j: (0, jnp.minimum(j, NB - 1))),
            pl.BlockSpec((T, BN, D_IN), lambda j: (0, jnp.maximum(j - NB, 0), 0)),
            pl.BlockSpec((T, D_IN), lambda j: (0, 0)),
        ],
        out_specs=[
            pl.BlockSpec((B, T, D_IN), lambda j: (0, 0, 0)),
            pl.BlockSpec((B, BN), lambda j: (0, jnp.maximum(j - NB, 0))),
        ],
        out_shape=[
            jax.ShapeDtypeStruct((B, T, D_IN), jnp.float32),
            jax.ShapeDtypeStruct((B, D_SAE), jnp.float32),
        ],
        scratch_shapes=[
            pltpu.VMEM((B, D_SAE), jnp.float32),
            pltpu.VMEM((B, 1), jnp.int32),
        ],
    )(x2, w_enc2, b_enc2, W_dec, b_dec)

    return (x_hat, z)
